# baseline (speedup 1.0000x reference)
import functools

import jax
import jax.numpy as jnp
from jax import lax
from jax.experimental import pallas as pl
from jax.experimental.pallas import tpu as pltpu
from jax.experimental.pallas import tpu_sc as plsc

NUM_HEROES = 1000000
EMBED_DIM = 32
BATCH = 16384
SEQ = 10

_info = plsc.get_sparse_core_info()
NC, NS, NL = _info.num_cores, _info.num_subcores, _info.num_lanes
NW = NC * NS
NBT = BATCH // 128
BT_PER_W = NBT // NW
NFT = EMBED_DIM // 8
SEQ_PAD = 16
TWORDS = NUM_HEROES * EMBED_DIM


def _body(x4_hbm, tf_hbm, out5, idx_c, ix0, ix1, gb0, gb1, g0, g1, w0, w1):
    wid = lax.axis_index("s") * NC + lax.axis_index("c")
    ixs = (ix0, ix1)
    gbs = (gb0, gb1)
    gsems = (g0, g1)
    wsems = (w0, w1)

    for c in range(BT_PER_W):
        bt = BT_PER_W * wid + c
        pltpu.sync_copy(x4_hbm.at[0, bt], idx_c.at[c, pl.ds(0, 8)])
        pltpu.sync_copy(x4_hbm.at[1, bt, pl.ds(0, 2)], idx_c.at[c, pl.ds(8, 2)])

    def gather(p):
        return pltpu.make_async_copy(tf_hbm.at[ixs[p]], gbs[p], gsems[p])

    def writes(c, s, p):
        bt = BT_PER_W * wid + c
        return [
            pltpu.make_async_copy(gbs[p].at[pl.ds(ft * 1024, 1024)],
                                  out5.at[s, ft, bt], wsems[p])
            for ft in range(NFT)
        ]

    NSTEP = BT_PER_W * SEQ

    def substep(i, p_):
        c = i // SEQ
        s = i - c * SEQ

        @pl.when(i >= 2)
        def _():
            c2 = (i - 2) // SEQ
            s2 = (i - 2) - c2 * SEQ
            for d in writes(c2, s2, p_):
                d.wait()

        for g in range(128 // NL):
            rv = idx_c[c, s, pl.ds(g * NL, NL)]
            for f in range(EMBED_DIM):
                ixs[p_][pl.ds(f * 128 + g * NL, NL)] = rv + f * NUM_HEROES

        gather(p_).start()

        @pl.when(i >= 1)
        def _():
            c1 = (i - 1) // SEQ
            s1 = (i - 1) - c1 * SEQ
            gather(1 - p_).wait()
            for d in writes(c1, s1, 1 - p_):
                d.start()

    def pair(j, carry):
        substep(2 * j, 0)
        substep(2 * j + 1, 1)
        return carry

    lax.fori_loop(0, NSTEP // 2, pair, 0)
    gather((NSTEP - 1) % 2).wait()
    for d in writes(BT_PER_W - 1, SEQ - 1, (NSTEP - 1) % 2):
        d.start()
    for d in writes(BT_PER_W - 1, SEQ - 2, (NSTEP - 2) % 2):
        d.wait()
    for d in writes(BT_PER_W - 1, SEQ - 1, (NSTEP - 1) % 2):
        d.wait()


@jax.jit
def kernel(x, table):
    xp = jnp.pad(x.T, ((0, SEQ_PAD - SEQ), (0, 0)))
    x4 = xp.reshape(2, 8, NBT, 128).transpose(0, 2, 1, 3)
    tflat = table.T.reshape(TWORDS)

    run = pl.kernel(
        _body,
        out_type=jax.ShapeDtypeStruct((SEQ, NFT, NBT, 1024), jnp.float32),
        mesh=plsc.VectorSubcoreMesh(core_axis_name="c", subcore_axis_name="s"),
        scratch_types=[
            pltpu.VMEM((BT_PER_W, SEQ, 128), jnp.int32),
            pltpu.VMEM((32 * 128,), jnp.int32),
            pltpu.VMEM((32 * 128,), jnp.int32),
            pltpu.VMEM((32 * 128,), jnp.float32),
            pltpu.VMEM((32 * 128,), jnp.float32),
            pltpu.SemaphoreType.DMA,
            pltpu.SemaphoreType.DMA,
            pltpu.SemaphoreType.DMA,
            pltpu.SemaphoreType.DMA,
        ],
        compiler_params=pltpu.CompilerParams(use_tc_tiling_on_sc=False),
    )
    out5 = run(x4, tflat)
    out = out5.reshape(SEQ, NFT, NBT, 8, 128)
    return out.transpose(2, 4, 0, 1, 3).reshape(BATCH, SEQ, EMBED_DIM)


# speedup vs baseline: 4.5422x; 4.5422x over previous
import functools

import jax
import jax.numpy as jnp
from jax import lax
from jax.experimental import pallas as pl
from jax.experimental.pallas import tpu as pltpu
from jax.experimental.pallas import tpu_sc as plsc

NUM_HEROES = 1000000
EMBED_DIM = 32
BATCH = 16384
SEQ = 10

_info = plsc.get_sparse_core_info()
NC, NS, NL = _info.num_cores, _info.num_subcores, _info.num_lanes
NW = NC * NS
NBT = BATCH // 128
BT_PER_W = NBT // NW
SEQ_PAD = 16


def _body(x4_hbm, table_hbm, out3, idx_c, buf0, buf1, g0, g1, w0, w1):
    wid = lax.axis_index("s") * NC + lax.axis_index("c")
    bufs = (buf0, buf1)
    gsems = (g0, g1)
    wsems = (w0, w1)

    def stage_idx(c):
        bt = BT_PER_W * wid + c
        pltpu.sync_copy(x4_hbm.at[0, bt], idx_c.at[c, pl.ds(0, 8)])
        pltpu.sync_copy(x4_hbm.at[1, bt, pl.ds(0, 2)], idx_c.at[c, pl.ds(8, 2)])

    def fire_gathers(c, b):
        return [
            pltpu.async_copy(table_hbm.at[idx_c.at[c, s]], bufs[b].at[s],
                             gsems[b])
            for s in range(SEQ)
        ]

    for c in range(BT_PER_W):
        stage_idx(c)

    gd = {}
    wd = {}
    for c in range(BT_PER_W + 1):
        if c < BT_PER_W:
            b = c % 2
            if c >= 2:
                wd[c - 2].wait()
            gd[c] = fire_gathers(c, b)
        if c >= 1:
            k = c - 1
            b = k % 2
            for d in gd[k]:
                d.wait()
            wd[k] = pltpu.async_copy(
                bufs[b],
                out3.at[:, pl.ds((BT_PER_W * wid + k) * 128, 128)],
                wsems[b])
    wd[BT_PER_W - 1].wait()
    wd[BT_PER_W - 2].wait()


@jax.jit
def kernel(x, table):
    xp = jnp.pad(x.T, ((0, SEQ_PAD - SEQ), (0, 0)))
    x4 = xp.reshape(2, 8, NBT, 128).transpose(0, 2, 1, 3)

    run = pl.kernel(
        _body,
        out_type=jax.ShapeDtypeStruct((SEQ, BATCH, EMBED_DIM), jnp.float32),
        mesh=plsc.VectorSubcoreMesh(core_axis_name="c", subcore_axis_name="s"),
        scratch_types=[
            pltpu.VMEM((BT_PER_W, SEQ, 128), jnp.int32),
            pltpu.VMEM((SEQ, 128, EMBED_DIM), jnp.float32),
            pltpu.VMEM((SEQ, 128, EMBED_DIM), jnp.float32),
            pltpu.SemaphoreType.DMA,
            pltpu.SemaphoreType.DMA,
            pltpu.SemaphoreType.DMA,
            pltpu.SemaphoreType.DMA,
        ],
        compiler_params=pltpu.CompilerParams(use_tc_tiling_on_sc=False),
    )
    out3 = run(x4, table)
    return out3.transpose(1, 0, 2)
